# baseline (speedup 1.0000x reference)
import dataclasses
import functools

import jax
import jax.numpy as jnp
from jax import lax
from jax.experimental import pallas as pl
from jax.experimental.pallas import tpu as pltpu
from jax.experimental.pallas import tpu_sc as plsc

VOCAB_SIZE = 32000
EMB_DIM = 256
HID_DIM = 512
OUT_DIM = 384
BATCH = 4096
SEQ = 200

NUM_CORES = 2
NUM_SUBCORES = 16
NUM_WORKERS = NUM_CORES * NUM_SUBCORES
SAMPLES_PER_WORKER = BATCH // NUM_WORKERS
LANES = 16
NUM_VREGS = EMB_DIM // LANES
GATHER_A = 128
GATHER_B = SEQ - GATHER_A

_mesh = plsc.VectorSubcoreMesh(core_axis_name="c", subcore_axis_name="s")

_sc_params = pltpu.CompilerParams()
if "needs_layout_passes" in pltpu.CompilerParams.__dataclass_fields__:
    _sc_params = dataclasses.replace(_sc_params, needs_layout_passes=False)


@functools.partial(
    pl.kernel,
    out_type=jax.ShapeDtypeStruct((BATCH, EMB_DIM), jnp.float32),
    mesh=_mesh,
    compiler_params=_sc_params,
    scratch_types=[
        pltpu.VMEM((SAMPLES_PER_WORKER * SEQ,), jnp.int32),
        pltpu.VMEM((SEQ, EMB_DIM // 2), jnp.int32),
        pltpu.VMEM((SEQ, EMB_DIM // 2), jnp.int32),
        pltpu.VMEM((SEQ, EMB_DIM // 2), jnp.int32),
        pltpu.VMEM((SEQ, EMB_DIM // 2), jnp.int32),
        pltpu.VMEM((EMB_DIM,), jnp.float32),
        pltpu.VMEM((EMB_DIM,), jnp.float32),
        pltpu.SemaphoreType.DMA,
        pltpu.SemaphoreType.DMA,
        pltpu.SemaphoreType.DMA,
        pltpu.SemaphoreType.DMA,
        pltpu.SemaphoreType.DMA,
        pltpu.SemaphoreType.DMA,
    ],
)
def _pool(tokens_hbm, table_hbm, out_hbm, idx_v, buf0, buf1, buf2, buf3,
          stage0, stage1, g0, g1, g2, g3, o0, o1):
    wid = lax.axis_index("s") * NUM_CORES + lax.axis_index("c")
    row_base = wid * SAMPLES_PER_WORKER

    tok_base = wid * (SAMPLES_PER_WORKER * SEQ)
    pltpu.sync_copy(tokens_hbm.at[pl.ds(tok_base, SAMPLES_PER_WORKER * SEQ)],
                    idx_v)

    def issue(s, buf, sem):
        off = pl.multiple_of(s * SEQ, 8)
        for lo, n in ((0, 56), (56, 48), (104, 48), (152, 48)):
            pltpu.async_copy(table_hbm.at[idx_v.at[pl.ds(off + lo, n)]],
                             buf.at[pl.ds(lo, n)], sem)

    def accumulate(buf, sem):
        def chunks(lo, hi, accs):
            ngrp = NUM_VREGS // 2

            def body(c, accs):
                base = c * 8
                b = [plsc.bitcast(buf[base, pl.ds(g * LANES, LANES)],
                                  jnp.bfloat16) for g in range(ngrp)]
                for k in range(1, 8):
                    for g in range(ngrp):
                        b[g] = b[g] + plsc.bitcast(
                            buf[base + k, pl.ds(g * LANES, LANES)],
                            jnp.bfloat16)
                a = list(accs)
                for g in range(ngrp):
                    w = plsc.bitcast(b[g], jnp.int32)
                    a[2 * g] = a[2 * g] + plsc.bitcast(w << 16, jnp.float32)
                    a[2 * g + 1] = a[2 * g + 1] + plsc.bitcast(
                        w & jnp.int32(-65536), jnp.float32)
                return tuple(a)
            return lax.fori_loop(lo // 8, hi // 8, body, accs)

        zeros = tuple(jnp.zeros((LANES,), jnp.float32)
                      for _ in range(NUM_VREGS))
        pltpu.make_async_copy(table_hbm.at[pl.ds(0, GATHER_A)],
                              buf.at[pl.ds(0, GATHER_A)], sem).wait()
        accs = chunks(0, GATHER_A, zeros)
        pltpu.make_async_copy(table_hbm.at[pl.ds(0, GATHER_B)],
                              buf.at[pl.ds(GATHER_A, GATHER_B)], sem).wait()
        return chunks(GATHER_A, SEQ, accs)

    def emit(s, accs, stage, sem):
        @pl.when(s >= 2)
        def _():
            pltpu.make_async_copy(stage, out_hbm.at[row_base], sem).wait()
        for g in range(NUM_VREGS // 2):
            stage[pl.ds(g * LANES, LANES)] = accs[2 * g]
            stage[pl.ds(EMB_DIM // 2 + g * LANES, LANES)] = accs[2 * g + 1]
        pltpu.async_copy(stage, out_hbm.at[row_base + s], sem)

    bufs = (buf0, buf1, buf2, buf3)
    gsems = (g0, g1, g2, g3)
    for k in range(4):
        issue(k, bufs[k], gsems[k])

    @pl.loop(0, SAMPLES_PER_WORKER, step=4)
    def _(s):
        for k in range(4):
            accs = accumulate(bufs[k], gsems[k])

            @pl.when(s + k + 4 < SAMPLES_PER_WORKER)
            def _():
                issue(s + k + 4, bufs[k], gsems[k])

            emit(s + k, accs, (stage0, stage1)[k % 2], (o0, o1)[k % 2])

    pltpu.make_async_copy(stage0, out_hbm.at[row_base], o0).wait()
    pltpu.make_async_copy(stage1, out_hbm.at[row_base], o1).wait()


def _pack_kernel(x_ref, o_ref):
    lo = x_ref[:, : EMB_DIM // 2].astype(jnp.bfloat16).astype(jnp.float32)
    hi = x_ref[:, EMB_DIM // 2:].astype(jnp.bfloat16).astype(jnp.float32)
    lo_bits = lax.bitcast_convert_type(lo, jnp.uint32) >> 16
    hi_bits = lax.bitcast_convert_type(hi, jnp.uint32) & jnp.uint32(0xFFFF0000)
    o_ref[...] = lax.bitcast_convert_type(lo_bits | hi_bits, jnp.int32)


_PACK_ROWS = 4000


def _pack_table(emb_table):
    return pl.pallas_call(
        _pack_kernel,
        grid=(VOCAB_SIZE // _PACK_ROWS,),
        in_specs=[pl.BlockSpec((_PACK_ROWS, EMB_DIM), lambda i: (i, 0))],
        out_specs=pl.BlockSpec((_PACK_ROWS, EMB_DIM // 2), lambda i: (i, 0)),
        out_shape=jax.ShapeDtypeStruct((VOCAB_SIZE, EMB_DIM // 2), jnp.int32),
    )(emb_table)


_SQRT_HALF = 0.7071067811865476


def _mlp_kernel(x_ref, w1_ref, b1_ref, w2_ref, b2_ref, o_ref):
    x = x_ref[...]
    h = jnp.dot(x, w1_ref[...], preferred_element_type=jnp.float32)
    h = h * (1.0 / SEQ) + b1_ref[...]
    h = 0.5 * h * (1.0 + lax.erf(h * _SQRT_HALF))
    o_ref[...] = jnp.dot(h, w2_ref[...],
                         preferred_element_type=jnp.float32) + b2_ref[...]


def kernel(tokens, emb_table, W1, b1, W2, b2):
    pooled_sum = _pool(tokens.reshape(-1).astype(jnp.int32),
                       _pack_table(emb_table))
    mlp_rows = BATCH // 4
    return pl.pallas_call(
        _mlp_kernel,
        grid=(BATCH // mlp_rows,),
        in_specs=[
            pl.BlockSpec((mlp_rows, EMB_DIM), lambda i: (i, 0)),
            pl.BlockSpec((EMB_DIM, HID_DIM), lambda i: (0, 0)),
            pl.BlockSpec((1, HID_DIM), lambda i: (0, 0)),
            pl.BlockSpec((HID_DIM, OUT_DIM), lambda i: (0, 0)),
            pl.BlockSpec((1, OUT_DIM), lambda i: (0, 0)),
        ],
        out_specs=pl.BlockSpec((mlp_rows, OUT_DIM), lambda i: (i, 0)),
        out_shape=jax.ShapeDtypeStruct((BATCH, OUT_DIM), jnp.float32),
    )(pooled_sum, W1, b1.reshape(1, HID_DIM), W2, b2.reshape(1, OUT_DIM))


# speedup vs baseline: 1.0019x; 1.0019x over previous
import dataclasses
import functools

import jax
import jax.numpy as jnp
from jax import lax
from jax.experimental import pallas as pl
from jax.experimental.pallas import tpu as pltpu
from jax.experimental.pallas import tpu_sc as plsc

VOCAB_SIZE = 32000
EMB_DIM = 256
HID_DIM = 512
OUT_DIM = 384
BATCH = 4096
SEQ = 200

NUM_CORES = 2
NUM_SUBCORES = 16
NUM_WORKERS = NUM_CORES * NUM_SUBCORES
SAMPLES_PER_WORKER = BATCH // NUM_WORKERS
LANES = 16
NUM_VREGS = EMB_DIM // LANES
GATHER_A = 128
GATHER_B = SEQ - GATHER_A

_mesh = plsc.VectorSubcoreMesh(core_axis_name="c", subcore_axis_name="s")

_sc_params = pltpu.CompilerParams()
if "needs_layout_passes" in pltpu.CompilerParams.__dataclass_fields__:
    _sc_params = dataclasses.replace(_sc_params, needs_layout_passes=False)


@functools.partial(
    pl.kernel,
    out_type=jax.ShapeDtypeStruct((BATCH, EMB_DIM), jnp.float32),
    mesh=_mesh,
    compiler_params=_sc_params,
    scratch_types=[
        pltpu.VMEM((SAMPLES_PER_WORKER * SEQ,), jnp.int32),
        pltpu.VMEM((SEQ, EMB_DIM // 2), jnp.int32),
        pltpu.VMEM((SEQ, EMB_DIM // 2), jnp.int32),
        pltpu.VMEM((SEQ, EMB_DIM // 2), jnp.int32),
        pltpu.VMEM((SEQ, EMB_DIM // 2), jnp.int32),
        pltpu.VMEM((EMB_DIM,), jnp.float32),
        pltpu.VMEM((EMB_DIM,), jnp.float32),
        pltpu.SemaphoreType.DMA,
        pltpu.SemaphoreType.DMA,
        pltpu.SemaphoreType.DMA,
        pltpu.SemaphoreType.DMA,
        pltpu.SemaphoreType.DMA,
        pltpu.SemaphoreType.DMA,
    ],
)
def _pool(tokens_hbm, table_hbm, out_hbm, idx_v, buf0, buf1, buf2, buf3,
          stage0, stage1, g0, g1, g2, g3, o0, o1):
    wid = lax.axis_index("s") * NUM_CORES + lax.axis_index("c")
    row_base = wid * SAMPLES_PER_WORKER

    tok_base = wid * (SAMPLES_PER_WORKER * SEQ)
    pltpu.sync_copy(tokens_hbm.at[pl.ds(tok_base, SAMPLES_PER_WORKER * SEQ)],
                    idx_v)

    def issue(s, buf, sem):
        off = pl.multiple_of(s * SEQ, 8)
        pltpu.async_copy(table_hbm.at[idx_v.at[pl.ds(off, GATHER_A)]],
                         buf.at[pl.ds(0, GATHER_A)], sem)
        pltpu.async_copy(
            table_hbm.at[idx_v.at[pl.ds(off + GATHER_A, GATHER_B)]],
            buf.at[pl.ds(GATHER_A, GATHER_B)], sem)

    def accumulate(buf, sem):
        def chunks(lo, hi, accs):
            ngrp = NUM_VREGS // 2

            def body(c, accs):
                base = c * 8
                b = [plsc.bitcast(buf[base, pl.ds(g * LANES, LANES)],
                                  jnp.bfloat16) for g in range(ngrp)]
                for k in range(1, 8):
                    for g in range(ngrp):
                        b[g] = b[g] + plsc.bitcast(
                            buf[base + k, pl.ds(g * LANES, LANES)],
                            jnp.bfloat16)
                a = list(accs)
                for g in range(ngrp):
                    w = plsc.bitcast(b[g], jnp.int32)
                    a[2 * g] = a[2 * g] + plsc.bitcast(w << 16, jnp.float32)
                    a[2 * g + 1] = a[2 * g + 1] + plsc.bitcast(
                        w & jnp.int32(-65536), jnp.float32)
                return tuple(a)
            return lax.fori_loop(lo // 8, hi // 8, body, accs)

        zeros = tuple(jnp.zeros((LANES,), jnp.float32)
                      for _ in range(NUM_VREGS))
        pltpu.make_async_copy(table_hbm.at[pl.ds(0, GATHER_A)],
                              buf.at[pl.ds(0, GATHER_A)], sem).wait()
        accs = chunks(0, GATHER_A, zeros)
        pltpu.make_async_copy(table_hbm.at[pl.ds(0, GATHER_B)],
                              buf.at[pl.ds(GATHER_A, GATHER_B)], sem).wait()
        return chunks(GATHER_A, SEQ, accs)

    def emit(s, accs, stage, sem):
        @pl.when(s >= 2)
        def _():
            pltpu.make_async_copy(stage, out_hbm.at[row_base], sem).wait()
        for g in range(NUM_VREGS // 2):
            stage[pl.ds(g * LANES, LANES)] = accs[2 * g]
            stage[pl.ds(EMB_DIM // 2 + g * LANES, LANES)] = accs[2 * g + 1]
        pltpu.async_copy(stage, out_hbm.at[row_base + s], sem)

    bufs = (buf0, buf1, buf2, buf3)
    gsems = (g0, g1, g2, g3)
    for k in range(4):
        issue(k, bufs[k], gsems[k])

    @pl.loop(0, SAMPLES_PER_WORKER, step=4)
    def _(s):
        for k in range(4):
            accs = accumulate(bufs[k], gsems[k])

            @pl.when(s + k + 4 < SAMPLES_PER_WORKER)
            def _():
                issue(s + k + 4, bufs[k], gsems[k])

            emit(s + k, accs, (stage0, stage1)[k % 2], (o0, o1)[k % 2])

    pltpu.make_async_copy(stage0, out_hbm.at[row_base], o0).wait()
    pltpu.make_async_copy(stage1, out_hbm.at[row_base], o1).wait()


def _pack_kernel(x_ref, o_ref):
    lo = x_ref[:, : EMB_DIM // 2].astype(jnp.bfloat16).astype(jnp.float32)
    hi = x_ref[:, EMB_DIM // 2:].astype(jnp.bfloat16).astype(jnp.float32)
    lo_bits = lax.bitcast_convert_type(lo, jnp.uint32) >> 16
    hi_bits = lax.bitcast_convert_type(hi, jnp.uint32) & jnp.uint32(0xFFFF0000)
    o_ref[...] = lax.bitcast_convert_type(lo_bits | hi_bits, jnp.int32)


_PACK_ROWS = 4000


def _pack_table(emb_table):
    return pl.pallas_call(
        _pack_kernel,
        grid=(VOCAB_SIZE // _PACK_ROWS,),
        in_specs=[pl.BlockSpec((_PACK_ROWS, EMB_DIM), lambda i: (i, 0))],
        out_specs=pl.BlockSpec((_PACK_ROWS, EMB_DIM // 2), lambda i: (i, 0)),
        out_shape=jax.ShapeDtypeStruct((VOCAB_SIZE, EMB_DIM // 2), jnp.int32),
    )(emb_table)


_SQRT_HALF = 0.7071067811865476


def _mlp_kernel(x_ref, w1_ref, b1_ref, w2_ref, b2_ref, o_ref):
    x = x_ref[...]
    h = jnp.dot(x, w1_ref[...], preferred_element_type=jnp.float32)
    h = h * (1.0 / SEQ) + b1_ref[...]
    h = 0.5 * h * (1.0 + lax.erf(h * _SQRT_HALF))
    o_ref[...] = jnp.dot(h, w2_ref[...],
                         preferred_element_type=jnp.float32) + b2_ref[...]


def kernel(tokens, emb_table, W1, b1, W2, b2):
    pooled_sum = _pool(tokens.reshape(-1).astype(jnp.int32),
                       _pack_table(emb_table))
    mlp_rows = BATCH // 4
    return pl.pallas_call(
        _mlp_kernel,
        grid=(BATCH // mlp_rows,),
        in_specs=[
            pl.BlockSpec((mlp_rows, EMB_DIM), lambda i: (i, 0)),
            pl.BlockSpec((EMB_DIM, HID_DIM), lambda i: (0, 0)),
            pl.BlockSpec((1, HID_DIM), lambda i: (0, 0)),
            pl.BlockSpec((HID_DIM, OUT_DIM), lambda i: (0, 0)),
            pl.BlockSpec((1, OUT_DIM), lambda i: (0, 0)),
        ],
        out_specs=pl.BlockSpec((mlp_rows, OUT_DIM), lambda i: (i, 0)),
        out_shape=jax.ShapeDtypeStruct((BATCH, OUT_DIM), jnp.float32),
    )(pooled_sum, W1, b1.reshape(1, HID_DIM), W2, b2.reshape(1, OUT_DIM))
